# baseline (device time: 12781 ns/iter reference)
import jax
import jax.numpy as jnp
from jax import lax
from jax.experimental import pallas as pl
from jax.experimental.pallas import tpu as pltpu

K = 16
NXY = 4


def _topk_desc(v, k):
    outs = []
    for _ in range(k):
        m = jnp.max(v, axis=1, keepdims=True)
        outs.append(m)
        v = jnp.where(v == m, -jnp.inf, v)
    return jnp.concatenate(outs, axis=1)


def _local_topk(v, k):
    _, n = v.shape
    w = 128
    m1 = v[:, :w]
    m2 = jnp.full_like(m1, -jnp.inf)
    for c in range(1, n // w):
        xc = v[:, c * w:(c + 1) * w]
        m2 = jnp.maximum(m2, jnp.minimum(m1, xc))
        m1 = jnp.maximum(m1, xc)
    return _topk_desc(jnp.concatenate([m1, m2], axis=1), k)


def kernel(x):
    rows, ncols = x.shape
    rblk = rows // NXY

    def body(xblk, out_ref, cand, send_sems, recv_sems):
        mx = lax.axis_index("x")
        my = lax.axis_index("y")
        mz = lax.axis_index("z")
        rb = 2 * mx + my
        r0 = rb * rblk

        barrier = pltpu.get_barrier_semaphore()
        for tx in (0, 1):
            for ty in (0, 1):
                for tz in (0, 1):
                    pl.semaphore_signal(
                        barrier, inc=1, device_id=(tx, ty, tz),
                        device_id_type=pl.DeviceIdType.MESH,
                    )

        tk = _local_topk(xblk[:, :], K)
        cand[mz, pl.ds(r0, rblk), :] = tk

        pl.semaphore_wait(barrier, 8)

        def for_each_peer(fn):
            for tx in (0, 1):
                for ty in (0, 1):
                    for tz in (0, 1):
                        not_self = jnp.logical_not(
                            (tx == mx) & (ty == my) & (tz == mz)
                        )
                        fn((tx, ty, tz), (tz, 2 * tx + ty), not_self)

        def start_send(target, tslot, not_self):
            @pl.when(not_self)
            def _():
                pltpu.make_async_remote_copy(
                    src_ref=cand.at[mz, pl.ds(r0, rblk), :],
                    dst_ref=cand.at[mz, pl.ds(r0, rblk), :],
                    send_sem=send_sems.at[tslot],
                    recv_sem=recv_sems.at[mz, rb],
                    device_id=target,
                    device_id_type=pl.DeviceIdType.MESH,
                ).start()

        for_each_peer(start_send)

        def recv_wait(z_, rb_):
            pltpu.make_async_remote_copy(
                src_ref=cand.at[mz, pl.ds(r0, rblk), :],
                dst_ref=cand.at[z_, pl.ds(rb_ * rblk, rblk), :],
                send_sem=send_sems.at[0, 0],
                recv_sem=recv_sems.at[z_, rb_],
                device_id=(mx, my, mz),
                device_id_type=pl.DeviceIdType.MESH,
            ).wait_recv()

        def merge_group(tx_):
            g0 = 2 * tx_ * rblk
            merged = jnp.concatenate(
                [cand[0, pl.ds(g0, 2 * rblk), :],
                 cand[1, pl.ds(g0, 2 * rblk), :]], axis=1
            )
            out_ref[pl.ds(g0, 2 * rblk), :] = _topk_desc(merged, K)

        for ty in (0, 1):
            for z in (0, 1):
                @pl.when(jnp.logical_not((ty == my) & (z == mz)))
                def _():
                    recv_wait(z, 2 * mx + ty)
        merge_group(mx)

        for ty in (0, 1):
            for z in (0, 1):
                recv_wait(z, 2 * (1 - mx) + ty)
        merge_group(1 - mx)

        def send_wait(target, tslot, not_self):
            @pl.when(not_self)
            def _():
                pltpu.make_async_remote_copy(
                    src_ref=cand.at[mz, pl.ds(r0, rblk), :],
                    dst_ref=cand.at[mz, pl.ds(r0, rblk), :],
                    send_sem=send_sems.at[tslot],
                    recv_sem=recv_sems.at[mz, rb],
                    device_id=target,
                    device_id_type=pl.DeviceIdType.MESH,
                ).wait_send()

        for_each_peer(send_wait)

    mx = lax.axis_index("x")
    my = lax.axis_index("y")
    r0 = (2 * mx + my) * rblk
    xblk = lax.dynamic_slice(x, (r0, 0), (rblk, ncols))

    return pl.pallas_call(
        body,
        out_shape=jax.ShapeDtypeStruct((rows, K), jnp.float32),
        in_specs=[pl.BlockSpec(memory_space=pltpu.VMEM)],
        out_specs=pl.BlockSpec(memory_space=pltpu.VMEM),
        scratch_shapes=[
            pltpu.VMEM((2, rows, K), jnp.float32),
            pltpu.SemaphoreType.DMA((2, NXY)),
            pltpu.SemaphoreType.DMA((2, NXY)),
        ],
        compiler_params=pltpu.CompilerParams(collective_id=0),
    )(xblk)


# device time: 11532 ns/iter; 1.1083x vs baseline; 1.1083x over previous
import jax
import jax.numpy as jnp
from jax import lax
from jax.experimental import pallas as pl
from jax.experimental.pallas import tpu as pltpu

K = 16
NXY = 4


def _topk_desc(v, k):
    outs = []
    for _ in range(k):
        m = jnp.max(v, axis=1, keepdims=True)
        outs.append(m)
        v = jnp.where(v == m, -jnp.inf, v)
    return jnp.concatenate(outs, axis=1)


def _local_topk(v, k):
    _, n = v.shape
    w = 128
    m1 = v[:, :w]
    m2 = jnp.full_like(m1, -jnp.inf)
    for c in range(1, n // w):
        xc = v[:, c * w:(c + 1) * w]
        m2 = jnp.maximum(m2, jnp.minimum(m1, xc))
        m1 = jnp.maximum(m1, xc)
    return _topk_desc(jnp.concatenate([m1, m2], axis=1), k)


def kernel(x):
    rows, ncols = x.shape
    rblk = rows // NXY

    def body(xblk, out_ref, cand, send_sems, recv_sems):
        mx = lax.axis_index("x")
        my = lax.axis_index("y")
        mz = lax.axis_index("z")
        rb = 2 * mx + my
        r0 = rb * rblk

        barrier = pltpu.get_barrier_semaphore()
        for tx in (0, 1):
            for ty in (0, 1):
                for tz in (0, 1):
                    pl.semaphore_signal(
                        barrier, inc=1, device_id=(tx, ty, tz),
                        device_id_type=pl.DeviceIdType.MESH,
                    )

        tk = _local_topk(xblk[:, :], K)
        cand[mz, pl.ds(r0, rblk), :] = tk

        pl.semaphore_wait(barrier, 8)

        def for_each_peer(fn):
            for tx in (0, 1):
                for ty in (0, 1):
                    for tz in (0, 1):
                        not_self = jnp.logical_not(
                            (tx == mx) & (ty == my) & (tz == mz)
                        )
                        fn((tx, ty, tz), (tz, 2 * tx + ty), not_self)

        def peer_rdma(target, sem_idx):
            return pltpu.make_async_remote_copy(
                src_ref=cand.at[mz, pl.ds(r0, rblk), :],
                dst_ref=cand.at[sem_idx[0], pl.ds(sem_idx[1] * rblk, rblk), :],
                send_sem=send_sems.at[sem_idx],
                recv_sem=recv_sems.at[sem_idx],
                device_id=target,
                device_id_type=pl.DeviceIdType.MESH,
            )

        def start_send(target, tslot, not_self):
            @pl.when(not_self)
            def _():
                pltpu.make_async_remote_copy(
                    src_ref=cand.at[mz, pl.ds(r0, rblk), :],
                    dst_ref=cand.at[mz, pl.ds(r0, rblk), :],
                    send_sem=send_sems.at[tslot],
                    recv_sem=recv_sems.at[mz, rb],
                    device_id=target,
                    device_id_type=pl.DeviceIdType.MESH,
                ).start()

        for_each_peer(start_send)

        def wait_recv(target, sslot, not_self):
            @pl.when(not_self)
            def _():
                peer_rdma(target, sslot).wait_recv()

        for_each_peer(wait_recv)

        merged = jnp.concatenate([cand[0], cand[1]], axis=1)
        out_ref[:, :] = _topk_desc(merged, K)

        def wait_send(target, tslot, not_self):
            @pl.when(not_self)
            def _():
                peer_rdma(target, tslot).wait_send()

        for_each_peer(wait_send)

    mx = lax.axis_index("x")
    my = lax.axis_index("y")
    r0 = (2 * mx + my) * rblk
    xblk = lax.dynamic_slice(x, (r0, 0), (rblk, ncols))

    return pl.pallas_call(
        body,
        out_shape=jax.ShapeDtypeStruct((rows, K), jnp.float32),
        in_specs=[pl.BlockSpec(memory_space=pltpu.VMEM)],
        out_specs=pl.BlockSpec(memory_space=pltpu.VMEM),
        scratch_shapes=[
            pltpu.VMEM((2, rows, K), jnp.float32),
            pltpu.SemaphoreType.DMA((2, NXY)),
            pltpu.SemaphoreType.DMA((2, NXY)),
        ],
        compiler_params=pltpu.CompilerParams(collective_id=0),
    )(xblk)
